# Initial kernel scaffold; baseline (speedup 1.0000x reference)
import functools

import jax
import jax.numpy as jnp
from jax import lax
from jax.experimental import pallas as pl
from jax.experimental.pallas import tpu as pltpu
from jax.experimental.pallas import tpu_sc as plsc

NC = 2
NT = 16
NW = NC * NT


def _elu(z):
    return jnp.where(z > 0, z, jnp.expm1(jnp.minimum(z, 0.0)))


def _ln(h, g, b, eps=1e-5):
    mu = jnp.mean(h, axis=-1, keepdims=True)
    var = jnp.mean((h - mu) ** 2, axis=-1, keepdims=True)
    return (h - mu) * lax.rsqrt(var + eps) * g + b


@functools.lru_cache(maxsize=None)
def _deg_kernel(N, E):
    ET = E // NW
    BD = 40
    KD = ET // BD
    RP = N // NT
    ZR = 125
    ZI = RP // ZR
    mesh = plsc.VectorSubcoreMesh(core_axis_name="c", subcore_axis_name="s")

    @functools.partial(
        pl.kernel,
        out_type=jax.ShapeDtypeStruct((NW, RP, 16), jnp.float32),
        mesh=mesh,
        scratch_types=[
            pltpu.VMEM((KD, BD), jnp.int32),
            pltpu.VMEM((BD, 16), jnp.float32),
            pltpu.VMEM((ZR, 16), jnp.float32),
            pltpu.VMEM_SHARED((N, 16), jnp.float32),
        ],
    )
    def k(dst_hbm, out_hbm, dst_v, ones_v, z_v, acc_sh):
        c = lax.axis_index("c")
        s = lax.axis_index("s")
        w = c * NT + s
        pltpu.sync_copy(dst_hbm.at[w], dst_v)

        def fill_ones(i, carry):
            ones_v[i, :] = jnp.ones((16,), jnp.float32)
            return carry

        lax.fori_loop(0, BD, fill_ones, 0)

        def fill_zero(i, carry):
            z_v[i, :] = jnp.zeros((16,), jnp.float32)
            return carry

        lax.fori_loop(0, ZR, fill_zero, 0)
        for z in range(ZI):
            pltpu.sync_copy(z_v, acc_sh.at[pl.ds(s * RP + z * ZR, ZR)])
        plsc.subcore_barrier()

        def body(j, carry):
            pltpu.sync_copy(ones_v, acc_sh.at[dst_v.at[j]], add=True)
            return carry

        lax.fori_loop(0, KD, body, 0)
        plsc.subcore_barrier()
        pltpu.sync_copy(acc_sh.at[pl.ds(s * RP, RP)], out_hbm.at[w])

    return k


@functools.lru_cache(maxsize=None)
def _scatter_kernel(N, E, Dh):
    B = 80
    ET = E // NT
    K = ET // B
    RP = N // NT
    ZR = 125
    ZI = RP // ZR
    mesh = plsc.VectorSubcoreMesh(core_axis_name="c", subcore_axis_name="s")

    @functools.partial(
        pl.kernel,
        out_type=jax.ShapeDtypeStruct((NW, RP, Dh), jnp.float32),
        mesh=mesh,
        scratch_types=[
            pltpu.VMEM((K, B), jnp.int32),
            pltpu.VMEM((K, B), jnp.int32),
            pltpu.VMEM((B, Dh), jnp.float32),
            pltpu.VMEM((B, Dh), jnp.float32),
            pltpu.VMEM((ZR, Dh), jnp.float32),
            pltpu.VMEM_SHARED((N, Dh), jnp.float32),
            pltpu.SemaphoreType.DMA,
            pltpu.SemaphoreType.DMA,
        ],
    )
    def k(g_hbm, srcs_hbm, dst_hbm, out_hbm, src_v, dst_v, r0, r1, z_v,
          acc_sh, sem0, sem1):
        c = lax.axis_index("c")
        s = lax.axis_index("s")
        w = c * NT + s
        pltpu.sync_copy(srcs_hbm.at[w], src_v)
        pltpu.sync_copy(dst_hbm.at[s], dst_v)

        def fill_zero(i, carry):
            for q in range(Dh // 16):
                z_v[i, pl.ds(q * 16, 16)] = jnp.zeros((16,), jnp.float32)
            return carry

        lax.fori_loop(0, ZR, fill_zero, 0)
        for z in range(ZI):
            pltpu.sync_copy(z_v, acc_sh.at[pl.ds(s * RP + z * ZR, ZR)])
        plsc.subcore_barrier()

        pltpu.async_copy(g_hbm.at[src_v.at[0]], r0, sem0)

        def body(jj, carry):
            j = 2 * jj
            pltpu.async_copy(g_hbm.at[src_v.at[j + 1]], r1, sem1)
            pltpu.make_async_copy(g_hbm.at[src_v.at[0]], r0, sem0).wait()
            pltpu.sync_copy(r0, acc_sh.at[dst_v.at[j]], add=True)
            pltpu.async_copy(g_hbm.at[src_v.at[j + 2]], r0, sem0)
            pltpu.make_async_copy(g_hbm.at[src_v.at[0]], r1, sem1).wait()
            pltpu.sync_copy(r1, acc_sh.at[dst_v.at[j + 1]], add=True)
            return carry

        lax.fori_loop(0, (K - 1) // 2, body, 0)
        pltpu.make_async_copy(g_hbm.at[src_v.at[0]], r0, sem0).wait()
        pltpu.sync_copy(r0, acc_sh.at[dst_v.at[K - 1]], add=True)

        plsc.subcore_barrier()
        pltpu.sync_copy(acc_sh.at[pl.ds(s * RP, RP)], out_hbm.at[w])

    return k


@functools.lru_cache(maxsize=None)
def _gather_kernel(N, Dp, NI):
    per = NI // NW
    mesh = plsc.VectorSubcoreMesh(core_axis_name="c", subcore_axis_name="s")

    @functools.partial(
        pl.kernel,
        out_type=jax.ShapeDtypeStruct((NI, Dp), jnp.float32),
        mesh=mesh,
        scratch_types=[
            pltpu.VMEM((per,), jnp.int32),
            pltpu.VMEM((per, Dp), jnp.float32),
            pltpu.SemaphoreType.DMA,
        ],
    )
    def k(p_hbm, idx_hbm, out_hbm, idx_v, rows_v, sem):
        c = lax.axis_index("c")
        s = lax.axis_index("s")
        w = c * NT + s
        pltpu.sync_copy(idx_hbm.at[w], idx_v)
        pltpu.async_copy(p_hbm.at[idx_v], rows_v, sem).wait()
        pltpu.sync_copy(rows_v, out_hbm.at[pl.ds(w * per, per)])

    return k


def _sc_deg(dst_r, N, E):
    return _deg_kernel(N, E)(dst_r)


def _sc_scatter(g_stack, srcs, dsts, N, E, Dh):
    return _scatter_kernel(N, E, Dh)(g_stack, srcs, dsts)


def _sc_gather(p, idxp, N, Dp, NI):
    return _gather_kernel(N, Dp, NI)(p, idxp)


_BN = 400


def _dot(a, b):
    return jnp.dot(a, b, preferred_element_type=jnp.float32,
                   precision=lax.Precision.HIGHEST)


def _tc_a(x, W1, dinv_col):
    N, IN = x.shape
    MID = W1.shape[1]
    H = MID // 2

    def body(x_ref, w_ref, d_ref, o_ref):
        m = _dot(x_ref[...], w_ref[...]) * d_ref[...]
        o_ref[0] = m[:, :H]
        o_ref[1] = m[:, H:]

    return pl.pallas_call(
        body,
        grid=(N // _BN,),
        in_specs=[
            pl.BlockSpec((_BN, IN), lambda i: (i, 0)),
            pl.BlockSpec((IN, MID), lambda i: (0, 0)),
            pl.BlockSpec((_BN, 1), lambda i: (i, 0)),
        ],
        out_specs=pl.BlockSpec((2, _BN, H), lambda i: (0, i, 0)),
        out_shape=jax.ShapeDtypeStruct((2, N, H), jnp.float32),
    )(x, W1, dinv_col)


def _tc_b(acc, g, dinv_col, b1, g1, bn1, W2):
    N = acc.shape[1]
    D = 2 * acc.shape[2]
    OUT = W2.shape[1]
    H = OUT // 2

    def body(a_ref, g_ref, d_ref, b_ref, gl_ref, bl_ref, w_ref, o_ref):
        a = jnp.concatenate([a_ref[0], a_ref[1]], axis=-1)
        gg = jnp.concatenate([g_ref[0], g_ref[1]], axis=-1)
        d = d_ref[...]
        z = (a + gg) * d + b_ref[...]
        h = _ln(_elu(z), gl_ref[...], bl_ref[...])
        m = _dot(h, w_ref[...]) * d
        o_ref[0] = m[:, :H]
        o_ref[1] = m[:, H:]

    return pl.pallas_call(
        body,
        grid=(N // _BN,),
        in_specs=[
            pl.BlockSpec((2, _BN, D // 2), lambda i: (0, i, 0)),
            pl.BlockSpec((2, _BN, D // 2), lambda i: (0, i, 0)),
            pl.BlockSpec((_BN, 1), lambda i: (i, 0)),
            pl.BlockSpec((D,), lambda i: (0,)),
            pl.BlockSpec((D,), lambda i: (0,)),
            pl.BlockSpec((D,), lambda i: (0,)),
            pl.BlockSpec((D, OUT), lambda i: (0, 0)),
        ],
        out_specs=pl.BlockSpec((2, _BN, H), lambda i: (0, i, 0)),
        out_shape=jax.ShapeDtypeStruct((2, N, H), jnp.float32),
    )(acc, g, dinv_col, b1, g1, bn1, W2)


def _tc_c(acc, g, dinv_col, b2, g2, bn2, fcWp, fcbp):
    N = acc.shape[1]
    D = 2 * acc.shape[2]
    P = fcWp.shape[1]
    H = D // 2

    def body(a_ref, g_ref, d_ref, b_ref, gl_ref, bl_ref, w_ref, fb_ref,
             h_ref, o_ref, p_ref):
        a = jnp.concatenate([a_ref[0], a_ref[1]], axis=-1)
        gg = jnp.concatenate([g_ref[0], g_ref[1]], axis=-1)
        d = d_ref[...]
        z = (a + gg) * d + b_ref[...]
        h = _ln(_elu(z), gl_ref[...], bl_ref[...])
        h_ref[...] = h
        m = h * d
        o_ref[0] = m[:, :H]
        o_ref[1] = m[:, H:]
        p_ref[...] = _dot(h, w_ref[...]) + fb_ref[...]

    return pl.pallas_call(
        body,
        grid=(N // _BN,),
        in_specs=[
            pl.BlockSpec((2, _BN, D // 2), lambda i: (0, i, 0)),
            pl.BlockSpec((2, _BN, D // 2), lambda i: (0, i, 0)),
            pl.BlockSpec((_BN, 1), lambda i: (i, 0)),
            pl.BlockSpec((D,), lambda i: (0,)),
            pl.BlockSpec((D,), lambda i: (0,)),
            pl.BlockSpec((D,), lambda i: (0,)),
            pl.BlockSpec((D, P), lambda i: (0, 0)),
            pl.BlockSpec((P,), lambda i: (0,)),
        ],
        out_specs=[
            pl.BlockSpec((_BN, D), lambda i: (i, 0)),
            pl.BlockSpec((2, _BN, H), lambda i: (0, i, 0)),
            pl.BlockSpec((_BN, P), lambda i: (i, 0)),
        ],
        out_shape=[
            jax.ShapeDtypeStruct((N, D), jnp.float32),
            jax.ShapeDtypeStruct((2, N, H), jnp.float32),
            jax.ShapeDtypeStruct((N, P), jnp.float32),
        ],
    )(acc, g, dinv_col, b2, g2, bn2, fcWp, fcbp)


def _tc_d(acc, g, dinv_col, W3, b3, g3, bn3, W4):
    N = acc.shape[1]
    D = 2 * acc.shape[2]
    MID = W3.shape[1]
    H = MID // 2

    def body(a_ref, g_ref, d_ref, w3_ref, b_ref, gl_ref, bl_ref, w4_ref,
             o_ref):
        a = jnp.concatenate([a_ref[0], a_ref[1]], axis=-1)
        gg = jnp.concatenate([g_ref[0], g_ref[1]], axis=-1)
        d = d_ref[...]
        u = (a + gg) * d
        z = _dot(u, w3_ref[...]) + b_ref[...]
        h = _ln(_elu(z), gl_ref[...], bl_ref[...])
        m = _dot(h, w4_ref[...]) * d
        o_ref[0] = m[:, :H]
        o_ref[1] = m[:, H:]

    return pl.pallas_call(
        body,
        grid=(N // _BN,),
        in_specs=[
            pl.BlockSpec((2, _BN, D // 2), lambda i: (0, i, 0)),
            pl.BlockSpec((2, _BN, D // 2), lambda i: (0, i, 0)),
            pl.BlockSpec((_BN, 1), lambda i: (i, 0)),
            pl.BlockSpec((D, MID), lambda i: (0, 0)),
            pl.BlockSpec((MID,), lambda i: (0,)),
            pl.BlockSpec((MID,), lambda i: (0,)),
            pl.BlockSpec((MID,), lambda i: (0,)),
            pl.BlockSpec((MID, MID), lambda i: (0, 0)),
        ],
        out_specs=pl.BlockSpec((2, _BN, H), lambda i: (0, i, 0)),
        out_shape=jax.ShapeDtypeStruct((2, N, H), jnp.float32),
    )(acc, g, dinv_col, W3, b3, g3, bn3, W4)


def _tc_e(acc, g, dinv_col, b4, g4, bn4):
    N = acc.shape[1]
    D = 2 * acc.shape[2]

    def body(a_ref, g_ref, d_ref, b_ref, gl_ref, bl_ref, h_ref):
        a = jnp.concatenate([a_ref[0], a_ref[1]], axis=-1)
        gg = jnp.concatenate([g_ref[0], g_ref[1]], axis=-1)
        z = (a + gg) * d_ref[...] + b_ref[...]
        h_ref[...] = _ln(_elu(z), gl_ref[...], bl_ref[...])

    return pl.pallas_call(
        body,
        grid=(N // _BN,),
        in_specs=[
            pl.BlockSpec((2, _BN, D // 2), lambda i: (0, i, 0)),
            pl.BlockSpec((2, _BN, D // 2), lambda i: (0, i, 0)),
            pl.BlockSpec((_BN, 1), lambda i: (i, 0)),
            pl.BlockSpec((D,), lambda i: (0,)),
            pl.BlockSpec((D,), lambda i: (0,)),
            pl.BlockSpec((D,), lambda i: (0,)),
        ],
        out_specs=pl.BlockSpec((_BN, D), lambda i: (i, 0)),
        out_shape=jax.ShapeDtypeStruct((N, D), jnp.float32),
    )(acc, g, dinv_col, b4, g4, bn4)


def kernel(x, edge_index, t, idx, W1, b1, g1, bn1, W2, b2, g2, bn2,
           W3, b3, g3, bn3, W4, b4, g4, bn4, fcW, fcb):
    N = x.shape[0]
    E = edge_index.shape[1]
    src = edge_index[0]
    dst = edge_index[1]

    dst_deg = dst.reshape(NW, -1, 40)
    deg16 = _sc_deg(dst_deg, N, E).reshape(NC, N, 16)
    deg = deg16[0, :, 0] + deg16[1, :, 0] + 1.0
    dinv_col = lax.rsqrt(deg).reshape(N, 1)

    src16 = src.reshape(NT, -1, 80)
    srcs = jnp.concatenate([src16, src16 + N], axis=0)
    dsts = dst.reshape(NT, -1, 80)

    def spmm(gsplit, Dh):
        acc = _sc_scatter(gsplit.reshape(2 * N, Dh), srcs, dsts, N, E, Dh)
        return acc.reshape(NC, N, Dh)

    g1s = _tc_a(x, W1, dinv_col)
    acc1 = spmm(g1s, W1.shape[1] // 2)
    g2s = _tc_b(acc1, g1s, dinv_col, b1, g1, bn1, W2)
    acc2 = spmm(g2s, W2.shape[1] // 2)
    fcWp = jnp.pad(fcW, ((0, 0), (0, 128 - fcW.shape[1])))
    fcbp = jnp.pad(fcb, (0, 128 - fcb.shape[0]))
    h2, g2bs, p = _tc_c(acc2, g2s, dinv_col, b2, g2, bn2, fcWp, fcbp)
    acc3 = spmm(g2bs, h2.shape[1] // 2)
    g4s = _tc_d(acc3, g2bs, dinv_col, W3, b3, g3, bn3, W4)
    acc4 = spmm(g4s, W4.shape[1] // 2)
    h4 = _tc_e(acc4, g4s, dinv_col, b4, g4, bn4)

    NI = 1024
    idxp = jnp.pad(idx, (0, NI - idx.shape[0])).reshape(NW, -1)
    rows = _sc_gather(p, idxp, N, p.shape[1], NI)
    class_prediction = rows[: idx.shape[0], : fcW.shape[1]]

    return (h2, h4, class_prediction)


# speedup vs baseline: 10.1013x; 10.1013x over previous
import functools

import jax
import jax.numpy as jnp
from jax import lax
from jax.experimental import pallas as pl
from jax.experimental.pallas import tpu as pltpu
from jax.experimental.pallas import tpu_sc as plsc

NC = 2
NT = 16
NW = NC * NT


def _elu(z):
    return jnp.where(z > 0, z, jnp.exp(jnp.minimum(z, 0.0)) - 1.0)


def _ln(h, g, b, eps=1e-5):
    mu = jnp.mean(h, axis=-1, keepdims=True)
    var = jnp.mean((h - mu) ** 2, axis=-1, keepdims=True)
    return (h - mu) * lax.rsqrt(var + eps) * g + b


@functools.lru_cache(maxsize=None)
def _deg_kernel(N, E):
    ET = E // NW
    BD = 40
    KD = ET // BD
    RP = N // NT
    ZR = 125
    ZI = RP // ZR
    mesh = plsc.VectorSubcoreMesh(core_axis_name="c", subcore_axis_name="s")

    @functools.partial(
        pl.kernel,
        out_type=jax.ShapeDtypeStruct((NW, RP, 16), jnp.float32),
        mesh=mesh,
        scratch_types=[
            pltpu.VMEM((KD, BD), jnp.int32),
            pltpu.VMEM((BD, 16), jnp.float32),
            pltpu.VMEM((ZR, 16), jnp.float32),
            pltpu.VMEM_SHARED((N, 16), jnp.float32),
        ],
    )
    def k(dst_hbm, out_hbm, dst_v, ones_v, z_v, acc_sh):
        c = lax.axis_index("c")
        s = lax.axis_index("s")
        w = c * NT + s
        pltpu.sync_copy(dst_hbm.at[w], dst_v)

        def fill_ones(i, carry):
            ones_v[i, :] = jnp.ones((16,), jnp.float32)
            return carry

        lax.fori_loop(0, BD, fill_ones, 0)

        def fill_zero(i, carry):
            z_v[i, :] = jnp.zeros((16,), jnp.float32)
            return carry

        lax.fori_loop(0, ZR, fill_zero, 0)
        for z in range(ZI):
            pltpu.sync_copy(z_v, acc_sh.at[pl.ds(s * RP + z * ZR, ZR)])
        plsc.subcore_barrier()

        def body(j, carry):
            pltpu.sync_copy(ones_v, acc_sh.at[dst_v.at[j]], add=True)
            return carry

        lax.fori_loop(0, KD, body, 0)
        plsc.subcore_barrier()
        pltpu.sync_copy(acc_sh.at[pl.ds(s * RP, RP)], out_hbm.at[w])

    return k


@functools.lru_cache(maxsize=None)
def _scatter_kernel(N, E, Dh):
    B = 80
    ET = E // NT
    K = ET // B
    RP = N // NT
    ZR = 125
    ZI = RP // ZR
    mesh = plsc.VectorSubcoreMesh(core_axis_name="c", subcore_axis_name="s")

    @functools.partial(
        pl.kernel,
        out_type=jax.ShapeDtypeStruct((NW, RP, Dh), jnp.float32),
        mesh=mesh,
        compiler_params=pltpu.CompilerParams(use_tc_tiling_on_sc=False),
        scratch_types=[
            pltpu.VMEM((K, B), jnp.int32),
            pltpu.VMEM((K, B), jnp.int32),
            pltpu.VMEM((B, Dh), jnp.float32),
            pltpu.VMEM((B, Dh), jnp.float32),
            pltpu.VMEM((ZR, Dh), jnp.float32),
            pltpu.VMEM_SHARED((N, Dh), jnp.float32),
            pltpu.SemaphoreType.DMA,
            pltpu.SemaphoreType.DMA,
        ],
    )
    def k(g_hbm, srcs_hbm, dst_hbm, out_hbm, src_v, dst_v, r0, r1, z_v,
          acc_sh, sem0, sem1):
        c = lax.axis_index("c")
        s = lax.axis_index("s")
        w = c * NT + s
        pltpu.sync_copy(srcs_hbm.at[w], src_v)
        pltpu.sync_copy(dst_hbm.at[s], dst_v)

        def fill_zero(i, carry):
            for q in range(Dh // 16):
                z_v[i, pl.ds(q * 16, 16)] = jnp.zeros((16,), jnp.float32)
            return carry

        lax.fori_loop(0, ZR, fill_zero, 0)
        for z in range(ZI):
            pltpu.sync_copy(z_v, acc_sh.at[pl.ds(s * RP + z * ZR, ZR)])
        plsc.subcore_barrier()

        pltpu.async_copy(g_hbm.at[src_v.at[0]], r0, sem0)

        def body(jj, carry):
            j = 2 * jj
            pltpu.async_copy(g_hbm.at[src_v.at[j + 1]], r1, sem1)
            pltpu.make_async_copy(g_hbm.at[src_v.at[0]], r0, sem0).wait()
            pltpu.sync_copy(r0, acc_sh.at[dst_v.at[j]], add=True)
            pltpu.async_copy(g_hbm.at[src_v.at[j + 2]], r0, sem0)
            pltpu.make_async_copy(g_hbm.at[src_v.at[0]], r1, sem1).wait()
            pltpu.sync_copy(r1, acc_sh.at[dst_v.at[j + 1]], add=True)
            return carry

        lax.fori_loop(0, (K - 1) // 2, body, 0)
        pltpu.make_async_copy(g_hbm.at[src_v.at[0]], r0, sem0).wait()
        pltpu.sync_copy(r0, acc_sh.at[dst_v.at[K - 1]], add=True)

        plsc.subcore_barrier()
        pltpu.sync_copy(acc_sh.at[pl.ds(s * RP, RP)], out_hbm.at[w])

    return k


@functools.lru_cache(maxsize=None)
def _gather_kernel(N, Dp, NI):
    per = NI // NW
    mesh = plsc.VectorSubcoreMesh(core_axis_name="c", subcore_axis_name="s")

    @functools.partial(
        pl.kernel,
        out_type=jax.ShapeDtypeStruct((NI, Dp), jnp.float32),
        mesh=mesh,
        scratch_types=[
            pltpu.VMEM((per,), jnp.int32),
            pltpu.VMEM((per, Dp), jnp.float32),
            pltpu.SemaphoreType.DMA,
        ],
    )
    def k(p_hbm, idx_hbm, out_hbm, idx_v, rows_v, sem):
        c = lax.axis_index("c")
        s = lax.axis_index("s")
        w = c * NT + s
        pltpu.sync_copy(idx_hbm.at[w], idx_v)
        pltpu.async_copy(p_hbm.at[idx_v], rows_v, sem).wait()
        pltpu.sync_copy(rows_v, out_hbm.at[pl.ds(w * per, per)])

    return k


def _sc_deg(dst_r, N, E):
    return _deg_kernel(N, E)(dst_r)


def _sc_scatter(g_stack, srcs, dsts, N, E, Dh):
    return _scatter_kernel(N, E, Dh)(g_stack, srcs, dsts)


def _sc_gather(p, idxp, N, Dp, NI):
    return _gather_kernel(N, Dp, NI)(p, idxp)


_BN = 400


def _dot(a, b):
    return jnp.dot(a, b, preferred_element_type=jnp.float32,
                   precision=lax.Precision.HIGHEST)


_DH = 64


def _split(m, o_ref):
    for q in range(o_ref.shape[0]):
        o_ref[q] = m[:, q * _DH:(q + 1) * _DH]


def _cat(ref):
    return jnp.concatenate([ref[q] for q in range(ref.shape[0])], axis=-1)


def _tc_a(x, W1, dinv_col):
    N, IN = x.shape
    MID = W1.shape[1]
    S = MID // _DH

    def body(x_ref, w_ref, d_ref, o_ref):
        m = _dot(x_ref[...], w_ref[...]) * d_ref[...]
        _split(m, o_ref)

    return pl.pallas_call(
        body,
        grid=(N // _BN,),
        in_specs=[
            pl.BlockSpec((_BN, IN), lambda i: (i, 0)),
            pl.BlockSpec((IN, MID), lambda i: (0, 0)),
            pl.BlockSpec((_BN, 1), lambda i: (i, 0)),
        ],
        out_specs=pl.BlockSpec((S, _BN, _DH), lambda i: (0, i, 0)),
        out_shape=jax.ShapeDtypeStruct((S, N, _DH), jnp.float32),
    )(x, W1, dinv_col)


def _tc_b(acc, g, dinv_col, b1, g1, bn1, W2):
    S = acc.shape[0]
    N = acc.shape[1]
    D = S * _DH
    OUT = W2.shape[1]
    SO = OUT // _DH

    def body(a_ref, g_ref, d_ref, b_ref, gl_ref, bl_ref, w_ref, o_ref):
        d = d_ref[...]
        z = (_cat(a_ref) + _cat(g_ref)) * d + b_ref[...]
        h = _ln(_elu(z), gl_ref[...], bl_ref[...])
        _split(_dot(h, w_ref[...]) * d, o_ref)

    return pl.pallas_call(
        body,
        grid=(N // _BN,),
        in_specs=[
            pl.BlockSpec((S, _BN, _DH), lambda i: (0, i, 0)),
            pl.BlockSpec((S, _BN, _DH), lambda i: (0, i, 0)),
            pl.BlockSpec((_BN, 1), lambda i: (i, 0)),
            pl.BlockSpec((D,), lambda i: (0,)),
            pl.BlockSpec((D,), lambda i: (0,)),
            pl.BlockSpec((D,), lambda i: (0,)),
            pl.BlockSpec((D, OUT), lambda i: (0, 0)),
        ],
        out_specs=pl.BlockSpec((SO, _BN, _DH), lambda i: (0, i, 0)),
        out_shape=jax.ShapeDtypeStruct((SO, N, _DH), jnp.float32),
    )(acc, g, dinv_col, b1, g1, bn1, W2)


def _tc_c(acc, g, dinv_col, b2, g2, bn2, fcWp, fcbp):
    S = acc.shape[0]
    N = acc.shape[1]
    D = S * _DH
    P = fcWp.shape[1]

    def body(a_ref, g_ref, d_ref, b_ref, gl_ref, bl_ref, w_ref, fb_ref,
             h_ref, o_ref, p_ref):
        d = d_ref[...]
        z = (_cat(a_ref) + _cat(g_ref)) * d + b_ref[...]
        h = _ln(_elu(z), gl_ref[...], bl_ref[...])
        h_ref[...] = h
        _split(h * d, o_ref)
        p_ref[...] = _dot(h, w_ref[...]) + fb_ref[...]

    return pl.pallas_call(
        body,
        grid=(N // _BN,),
        in_specs=[
            pl.BlockSpec((S, _BN, _DH), lambda i: (0, i, 0)),
            pl.BlockSpec((S, _BN, _DH), lambda i: (0, i, 0)),
            pl.BlockSpec((_BN, 1), lambda i: (i, 0)),
            pl.BlockSpec((D,), lambda i: (0,)),
            pl.BlockSpec((D,), lambda i: (0,)),
            pl.BlockSpec((D,), lambda i: (0,)),
            pl.BlockSpec((D, P), lambda i: (0, 0)),
            pl.BlockSpec((P,), lambda i: (0,)),
        ],
        out_specs=[
            pl.BlockSpec((_BN, D), lambda i: (i, 0)),
            pl.BlockSpec((S, _BN, _DH), lambda i: (0, i, 0)),
            pl.BlockSpec((_BN, P), lambda i: (i, 0)),
        ],
        out_shape=[
            jax.ShapeDtypeStruct((N, D), jnp.float32),
            jax.ShapeDtypeStruct((S, N, _DH), jnp.float32),
            jax.ShapeDtypeStruct((N, P), jnp.float32),
        ],
    )(acc, g, dinv_col, b2, g2, bn2, fcWp, fcbp)


def _tc_d(acc, g, dinv_col, W3, b3, g3, bn3, W4):
    S = acc.shape[0]
    N = acc.shape[1]
    D = S * _DH
    MID = W3.shape[1]
    SO = W4.shape[1] // _DH

    def body(a_ref, g_ref, d_ref, w3_ref, b_ref, gl_ref, bl_ref, w4_ref,
             o_ref):
        d = d_ref[...]
        u = (_cat(a_ref) + _cat(g_ref)) * d
        z = _dot(u, w3_ref[...]) + b_ref[...]
        h = _ln(_elu(z), gl_ref[...], bl_ref[...])
        _split(_dot(h, w4_ref[...]) * d, o_ref)

    return pl.pallas_call(
        body,
        grid=(N // _BN,),
        in_specs=[
            pl.BlockSpec((S, _BN, _DH), lambda i: (0, i, 0)),
            pl.BlockSpec((S, _BN, _DH), lambda i: (0, i, 0)),
            pl.BlockSpec((_BN, 1), lambda i: (i, 0)),
            pl.BlockSpec((D, MID), lambda i: (0, 0)),
            pl.BlockSpec((MID,), lambda i: (0,)),
            pl.BlockSpec((MID,), lambda i: (0,)),
            pl.BlockSpec((MID,), lambda i: (0,)),
            pl.BlockSpec((MID, W4.shape[1]), lambda i: (0, 0)),
        ],
        out_specs=pl.BlockSpec((SO, _BN, _DH), lambda i: (0, i, 0)),
        out_shape=jax.ShapeDtypeStruct((SO, N, _DH), jnp.float32),
    )(acc, g, dinv_col, W3, b3, g3, bn3, W4)


def _tc_e(acc, g, dinv_col, b4, g4, bn4):
    S = acc.shape[0]
    N = acc.shape[1]
    D = S * _DH

    def body(a_ref, g_ref, d_ref, b_ref, gl_ref, bl_ref, h_ref):
        z = (_cat(a_ref) + _cat(g_ref)) * d_ref[...] + b_ref[...]
        h_ref[...] = _ln(_elu(z), gl_ref[...], bl_ref[...])

    return pl.pallas_call(
        body,
        grid=(N // _BN,),
        in_specs=[
            pl.BlockSpec((S, _BN, _DH), lambda i: (0, i, 0)),
            pl.BlockSpec((S, _BN, _DH), lambda i: (0, i, 0)),
            pl.BlockSpec((_BN, 1), lambda i: (i, 0)),
            pl.BlockSpec((D,), lambda i: (0,)),
            pl.BlockSpec((D,), lambda i: (0,)),
            pl.BlockSpec((D,), lambda i: (0,)),
        ],
        out_specs=pl.BlockSpec((_BN, D), lambda i: (i, 0)),
        out_shape=jax.ShapeDtypeStruct((N, D), jnp.float32),
    )(acc, g, dinv_col, b4, g4, bn4)


def kernel(x, edge_index, t, idx, W1, b1, g1, bn1, W2, b2, g2, bn2,
           W3, b3, g3, bn3, W4, b4, g4, bn4, fcW, fcb):
    N = x.shape[0]
    E = edge_index.shape[1]
    src = edge_index[0]
    dst = edge_index[1]

    dst_deg = dst.reshape(NW, -1, 40)
    deg16 = _sc_deg(dst_deg, N, E).reshape(NC, N, 16)
    deg = deg16[0, :, 0] + deg16[1, :, 0] + 1.0
    dinv_col = lax.rsqrt(deg).reshape(N, 1)

    src16 = src.reshape(NT, -1, 80)
    srcs = jnp.concatenate([src16, src16 + N], axis=0)
    dsts = dst.reshape(NT, -1, 80)

    def spmm(gq):
        S = gq.shape[0]
        accs = [
            _sc_scatter(gq[q:q + 2].reshape(2 * N, _DH), srcs, dsts, N, E,
                        _DH).reshape(NC, N, _DH)
            for q in range(0, S, 2)
        ]
        return jnp.concatenate(accs, axis=0)

    g1s = _tc_a(x, W1, dinv_col)
    acc1 = spmm(g1s)
    g2s = _tc_b(acc1, g1s, dinv_col, b1, g1, bn1, W2)
    acc2 = spmm(g2s)
    fcWp = jnp.pad(fcW, ((0, 0), (0, 128 - fcW.shape[1])))
    fcbp = jnp.pad(fcb, (0, 128 - fcb.shape[0]))
    h2, g2bs, p = _tc_c(acc2, g2s, dinv_col, b2, g2, bn2, fcWp, fcbp)
    acc3 = spmm(g2bs)
    g4s = _tc_d(acc3, g2bs, dinv_col, W3, b3, g3, bn3, W4)
    acc4 = spmm(g4s)
    h4 = _tc_e(acc4, g4s, dinv_col, b4, g4, bn4)

    NI = 1024
    idxp = jnp.pad(idx, (0, NI - idx.shape[0])).reshape(NW, -1)
    rows = _sc_gather(p, idxp, N, p.shape[1], NI)
    class_prediction = rows[: idx.shape[0], : fcW.shape[1]]

    return (h2, h4, class_prediction)


# speedup vs baseline: 10.5024x; 1.0397x over previous
import functools

import jax
import jax.numpy as jnp
from jax import lax
from jax.experimental import pallas as pl
from jax.experimental.pallas import tpu as pltpu
from jax.experimental.pallas import tpu_sc as plsc

NC = 2
NT = 16
NW = NC * NT


def _elu(z):
    return jnp.where(z > 0, z, jnp.exp(jnp.minimum(z, 0.0)) - 1.0)


def _ln(h, g, b, eps=1e-5):
    mu = jnp.mean(h, axis=-1, keepdims=True)
    var = jnp.mean((h - mu) ** 2, axis=-1, keepdims=True)
    return (h - mu) * lax.rsqrt(var + eps) * g + b


@functools.lru_cache(maxsize=None)
def _deg_kernel(N, E):
    ET = E // NW
    BD = 40
    KD = ET // BD
    RP = N // NT
    ZR = 125
    ZI = RP // ZR
    mesh = plsc.VectorSubcoreMesh(core_axis_name="c", subcore_axis_name="s")

    @functools.partial(
        pl.kernel,
        out_type=jax.ShapeDtypeStruct((NW, RP, 16), jnp.float32),
        mesh=mesh,
        scratch_types=[
            pltpu.VMEM((KD, BD), jnp.int32),
            pltpu.VMEM((BD, 16), jnp.float32),
            pltpu.VMEM((ZR, 16), jnp.float32),
            pltpu.VMEM_SHARED((N, 16), jnp.float32),
        ],
    )
    def k(dst_hbm, out_hbm, dst_v, ones_v, z_v, acc_sh):
        c = lax.axis_index("c")
        s = lax.axis_index("s")
        w = c * NT + s
        pltpu.sync_copy(dst_hbm.at[w], dst_v)

        def fill_ones(i, carry):
            ones_v[i, :] = jnp.ones((16,), jnp.float32)
            return carry

        lax.fori_loop(0, BD, fill_ones, 0)

        def fill_zero(i, carry):
            z_v[i, :] = jnp.zeros((16,), jnp.float32)
            return carry

        lax.fori_loop(0, ZR, fill_zero, 0)
        for z in range(ZI):
            pltpu.sync_copy(z_v, acc_sh.at[pl.ds(s * RP + z * ZR, ZR)])
        plsc.subcore_barrier()

        def body(j, carry):
            pltpu.sync_copy(ones_v, acc_sh.at[dst_v.at[j]], add=True)
            return carry

        lax.fori_loop(0, KD, body, 0)
        plsc.subcore_barrier()
        pltpu.sync_copy(acc_sh.at[pl.ds(s * RP, RP)], out_hbm.at[w])

    return k


@functools.lru_cache(maxsize=None)
def _scatter_kernel(N, E, Dh, NP, NI):
    B = 80
    ET = E // NT
    K = ET // B
    RP = N // NT
    ZR = 125
    ZI = RP // ZR
    GP = NI // NW if NI else 0
    mesh = plsc.VectorSubcoreMesh(core_axis_name="c", subcore_axis_name="s")

    out_types = [jax.ShapeDtypeStruct((NP * NW, RP, Dh), jnp.float32)]
    scratch = [
        pltpu.VMEM((K, B), jnp.int32),
        pltpu.VMEM((K, B), jnp.int32),
        pltpu.VMEM((B, Dh), jnp.float32),
        pltpu.VMEM((B, Dh), jnp.float32),
        pltpu.VMEM((ZR, Dh), jnp.float32),
        pltpu.VMEM_SHARED((N, Dh), jnp.float32),
        pltpu.SemaphoreType.DMA,
        pltpu.SemaphoreType.DMA,
    ]
    if NI:
        out_types.append(jax.ShapeDtypeStruct((NI, 2 * Dh), jnp.float32))
        scratch += [pltpu.VMEM((GP,), jnp.int32),
                    pltpu.VMEM((GP, 2 * Dh), jnp.float32)]

    @functools.partial(
        pl.kernel,
        out_type=tuple(out_types) if NI else out_types[0],
        mesh=mesh,
        compiler_params=pltpu.CompilerParams(use_tc_tiling_on_sc=False),
        scratch_types=scratch,
    )
    def k(*refs):
        if NI:
            (g_hbm, srcs_hbm, dst_hbm, ptab_hbm, idx_hbm, out_hbm, rows_hbm,
             src_v, dst_v, r0, r1, z_v, acc_sh, sem0, sem1, idx_v,
             prow_v) = refs
        else:
            (g_hbm, srcs_hbm, dst_hbm, out_hbm,
             src_v, dst_v, r0, r1, z_v, acc_sh, sem0, sem1) = refs
        c = lax.axis_index("c")
        s = lax.axis_index("s")
        w = c * NT + s
        pltpu.sync_copy(dst_hbm.at[s], dst_v)

        def fill_zero(i, carry):
            for q in range(Dh // 16):
                z_v[i, pl.ds(q * 16, 16)] = jnp.zeros((16,), jnp.float32)
            return carry

        lax.fori_loop(0, ZR, fill_zero, 0)

        for p in range(NP):
            pltpu.sync_copy(srcs_hbm.at[p * NW + w], src_v)
            for z in range(ZI):
                pltpu.sync_copy(z_v, acc_sh.at[pl.ds(s * RP + z * ZR, ZR)])
            plsc.subcore_barrier()

            pltpu.async_copy(g_hbm.at[src_v.at[0]], r0, sem0)

            def body(jj, carry):
                j = 2 * jj
                pltpu.async_copy(g_hbm.at[src_v.at[j + 1]], r1, sem1)
                pltpu.make_async_copy(g_hbm.at[src_v.at[0]], r0, sem0).wait()
                pltpu.sync_copy(r0, acc_sh.at[dst_v.at[j]], add=True)
                pltpu.async_copy(g_hbm.at[src_v.at[j + 2]], r0, sem0)
                pltpu.make_async_copy(g_hbm.at[src_v.at[0]], r1, sem1).wait()
                pltpu.sync_copy(r1, acc_sh.at[dst_v.at[j + 1]], add=True)
                return carry

            lax.fori_loop(0, (K - 1) // 2, body, 0)
            pltpu.make_async_copy(g_hbm.at[src_v.at[0]], r0, sem0).wait()
            pltpu.sync_copy(r0, acc_sh.at[dst_v.at[K - 1]], add=True)

            plsc.subcore_barrier()
            pltpu.sync_copy(acc_sh.at[pl.ds(s * RP, RP)],
                            out_hbm.at[p * NW + w])
            if p + 1 < NP:
                plsc.subcore_barrier()

        if NI:
            pltpu.sync_copy(idx_hbm.at[w], idx_v)
            pltpu.async_copy(ptab_hbm.at[idx_v], prow_v, sem0).wait()
            pltpu.sync_copy(prow_v, rows_hbm.at[pl.ds(w * GP, GP)])

    return k


def _sc_deg(dst_r, N, E):
    return _deg_kernel(N, E)(dst_r)


def _sc_spmm1(gq, srcs2, dsts, N, E):
    Dh = gq.shape[2]
    out = _scatter_kernel(N, E, Dh, 1, 0)(gq.reshape(2 * N, Dh), srcs2, dsts)
    return out.reshape(NC, N, Dh)


def _sc_spmm2(gq, srcs4, dsts, ptab, idxp, N, E, NI):
    Dh = gq.shape[2]
    out, rows = _scatter_kernel(N, E, Dh, 2, NI)(
        gq.reshape(4 * N, Dh), srcs4, dsts, ptab, idxp)
    return out.reshape(2 * NC, N, Dh), rows


_BN = 400


def _dot(a, b):
    return jnp.dot(a, b, preferred_element_type=jnp.float32,
                   precision=lax.Precision.HIGHEST)


_DH = 64


def _split(m, o_ref):
    for q in range(o_ref.shape[0]):
        o_ref[q] = m[:, q * _DH:(q + 1) * _DH]


def _cat(ref):
    return jnp.concatenate([ref[q] for q in range(ref.shape[0])], axis=-1)


def _tc_a(x, W1, dinv_col):
    N, IN = x.shape
    MID = W1.shape[1]
    S = MID // _DH

    def body(x_ref, w_ref, d_ref, o_ref):
        m = _dot(x_ref[...], w_ref[...]) * d_ref[...]
        _split(m, o_ref)

    return pl.pallas_call(
        body,
        grid=(N // _BN,),
        in_specs=[
            pl.BlockSpec((_BN, IN), lambda i: (i, 0)),
            pl.BlockSpec((IN, MID), lambda i: (0, 0)),
            pl.BlockSpec((_BN, 1), lambda i: (i, 0)),
        ],
        out_specs=pl.BlockSpec((S, _BN, _DH), lambda i: (0, i, 0)),
        out_shape=jax.ShapeDtypeStruct((S, N, _DH), jnp.float32),
    )(x, W1, dinv_col)


def _tc_b(acc, g, dinv_col, b1, g1, bn1, W2):
    S = acc.shape[0]
    N = acc.shape[1]
    D = S * _DH
    OUT = W2.shape[1]
    SO = OUT // _DH

    def body(a_ref, g_ref, d_ref, b_ref, gl_ref, bl_ref, w_ref, o_ref):
        d = d_ref[...]
        z = (_cat(a_ref) + _cat(g_ref)) * d + b_ref[...]
        h = _ln(_elu(z), gl_ref[...], bl_ref[...])
        _split(_dot(h, w_ref[...]) * d, o_ref)

    return pl.pallas_call(
        body,
        grid=(N // _BN,),
        in_specs=[
            pl.BlockSpec((S, _BN, _DH), lambda i: (0, i, 0)),
            pl.BlockSpec((S, _BN, _DH), lambda i: (0, i, 0)),
            pl.BlockSpec((_BN, 1), lambda i: (i, 0)),
            pl.BlockSpec((D,), lambda i: (0,)),
            pl.BlockSpec((D,), lambda i: (0,)),
            pl.BlockSpec((D,), lambda i: (0,)),
            pl.BlockSpec((D, OUT), lambda i: (0, 0)),
        ],
        out_specs=pl.BlockSpec((SO, _BN, _DH), lambda i: (0, i, 0)),
        out_shape=jax.ShapeDtypeStruct((SO, N, _DH), jnp.float32),
    )(acc, g, dinv_col, b1, g1, bn1, W2)


def _tc_c(acc, g, dinv_col, b2, g2, bn2, fcWp, fcbp):
    S = acc.shape[0]
    N = acc.shape[1]
    D = S * _DH
    P = fcWp.shape[1]

    def body(a_ref, g_ref, d_ref, b_ref, gl_ref, bl_ref, w_ref, fb_ref,
             h_ref, o_ref, p_ref):
        d = d_ref[...]
        z = (_cat(a_ref) + _cat(g_ref)) * d + b_ref[...]
        h = _ln(_elu(z), gl_ref[...], bl_ref[...])
        h_ref[...] = h
        _split(h * d, o_ref)
        p_ref[...] = _dot(h, w_ref[...]) + fb_ref[...]

    return pl.pallas_call(
        body,
        grid=(N // _BN,),
        in_specs=[
            pl.BlockSpec((S, _BN, _DH), lambda i: (0, i, 0)),
            pl.BlockSpec((S, _BN, _DH), lambda i: (0, i, 0)),
            pl.BlockSpec((_BN, 1), lambda i: (i, 0)),
            pl.BlockSpec((D,), lambda i: (0,)),
            pl.BlockSpec((D,), lambda i: (0,)),
            pl.BlockSpec((D,), lambda i: (0,)),
            pl.BlockSpec((D, P), lambda i: (0, 0)),
            pl.BlockSpec((P,), lambda i: (0,)),
        ],
        out_specs=[
            pl.BlockSpec((_BN, D), lambda i: (i, 0)),
            pl.BlockSpec((S, _BN, _DH), lambda i: (0, i, 0)),
            pl.BlockSpec((_BN, P), lambda i: (i, 0)),
        ],
        out_shape=[
            jax.ShapeDtypeStruct((N, D), jnp.float32),
            jax.ShapeDtypeStruct((S, N, _DH), jnp.float32),
            jax.ShapeDtypeStruct((N, P), jnp.float32),
        ],
    )(acc, g, dinv_col, b2, g2, bn2, fcWp, fcbp)


def _tc_d(acc, g, dinv_col, W3, b3, g3, bn3, W4):
    S = acc.shape[0]
    N = acc.shape[1]
    D = S * _DH
    MID = W3.shape[1]
    SO = W4.shape[1] // _DH

    def body(a_ref, g_ref, d_ref, w3_ref, b_ref, gl_ref, bl_ref, w4_ref,
             o_ref):
        d = d_ref[...]
        u = (_cat(a_ref) + _cat(g_ref)) * d
        z = _dot(u, w3_ref[...]) + b_ref[...]
        h = _ln(_elu(z), gl_ref[...], bl_ref[...])
        _split(_dot(h, w4_ref[...]) * d, o_ref)

    return pl.pallas_call(
        body,
        grid=(N // _BN,),
        in_specs=[
            pl.BlockSpec((S, _BN, _DH), lambda i: (0, i, 0)),
            pl.BlockSpec((S, _BN, _DH), lambda i: (0, i, 0)),
            pl.BlockSpec((_BN, 1), lambda i: (i, 0)),
            pl.BlockSpec((D, MID), lambda i: (0, 0)),
            pl.BlockSpec((MID,), lambda i: (0,)),
            pl.BlockSpec((MID,), lambda i: (0,)),
            pl.BlockSpec((MID,), lambda i: (0,)),
            pl.BlockSpec((MID, W4.shape[1]), lambda i: (0, 0)),
        ],
        out_specs=pl.BlockSpec((SO, _BN, _DH), lambda i: (0, i, 0)),
        out_shape=jax.ShapeDtypeStruct((SO, N, _DH), jnp.float32),
    )(acc, g, dinv_col, W3, b3, g3, bn3, W4)


def _tc_e(acc, g, dinv_col, b4, g4, bn4):
    S = acc.shape[0]
    N = acc.shape[1]
    D = S * _DH

    def body(a_ref, g_ref, d_ref, b_ref, gl_ref, bl_ref, h_ref):
        z = (_cat(a_ref) + _cat(g_ref)) * d_ref[...] + b_ref[...]
        h_ref[...] = _ln(_elu(z), gl_ref[...], bl_ref[...])

    return pl.pallas_call(
        body,
        grid=(N // _BN,),
        in_specs=[
            pl.BlockSpec((S, _BN, _DH), lambda i: (0, i, 0)),
            pl.BlockSpec((S, _BN, _DH), lambda i: (0, i, 0)),
            pl.BlockSpec((_BN, 1), lambda i: (i, 0)),
            pl.BlockSpec((D,), lambda i: (0,)),
            pl.BlockSpec((D,), lambda i: (0,)),
            pl.BlockSpec((D,), lambda i: (0,)),
        ],
        out_specs=pl.BlockSpec((_BN, D), lambda i: (i, 0)),
        out_shape=jax.ShapeDtypeStruct((N, D), jnp.float32),
    )(acc, g, dinv_col, b4, g4, bn4)


def kernel(x, edge_index, t, idx, W1, b1, g1, bn1, W2, b2, g2, bn2,
           W3, b3, g3, bn3, W4, b4, g4, bn4, fcW, fcb):
    N = x.shape[0]
    E = edge_index.shape[1]
    src = edge_index[0]
    dst = edge_index[1]

    dst_deg = dst.reshape(NW, -1, 40)
    deg16 = _sc_deg(dst_deg, N, E).reshape(NC, N, 16)
    deg = deg16[0, :, 0] + deg16[1, :, 0] + 1.0
    dinv_col = lax.rsqrt(deg).reshape(N, 1)

    src16 = src.reshape(NT, -1, 80)
    srcs2 = jnp.concatenate([src16, src16 + N], axis=0)
    srcs4 = jnp.concatenate(
        [src16, src16 + N, src16 + 2 * N, src16 + 3 * N], axis=0)
    dsts = dst.reshape(NT, -1, 80)
    NI = 1024
    idxp = jnp.pad(idx, (0, NI - idx.shape[0])).reshape(NW, -1)

    g1s = _tc_a(x, W1, dinv_col)
    dummy_p = jnp.zeros((N, 2 * _DH), jnp.float32)
    acc1, _ = _sc_spmm2(g1s, srcs4, dsts, dummy_p, idxp, N, E, NI)
    g2s = _tc_b(acc1, g1s, dinv_col, b1, g1, bn1, W2)
    acc2 = _sc_spmm1(g2s, srcs2, dsts, N, E)
    fcWp = jnp.pad(fcW, ((0, 0), (0, 128 - fcW.shape[1])))
    fcbp = jnp.pad(fcb, (0, 128 - fcb.shape[0]))
    h2, g2bs, p = _tc_c(acc2, g2s, dinv_col, b2, g2, bn2, fcWp, fcbp)
    acc3 = _sc_spmm1(g2bs, srcs2, dsts, N, E)
    g4s = _tc_d(acc3, g2bs, dinv_col, W3, b3, g3, bn3, W4)
    acc4, rows = _sc_spmm2(g4s, srcs4, dsts, p, idxp, N, E, NI)
    h4 = _tc_e(acc4, g4s, dinv_col, b4, g4, bn4)

    class_prediction = rows[: idx.shape[0], : fcW.shape[1]]

    return (h2, h4, class_prediction)
